# Initial kernel scaffold; baseline (speedup 1.0000x reference)
import jax
import jax.numpy as jnp
from jax.experimental import pallas as pl


def kernel(x, edge_index, edge_attr, batch, node_W, node_b, bond_W, bond_b, conv_W1, conv_b1, conv_W2, conv_b2, eps, bn_gamma, bn_beta, pred_W, pred_b):
    raise NotImplementedError("write your pallas kernel here")


# speedup vs baseline: 2.6973x; 2.6973x over previous
import functools

import jax
import jax.numpy as jnp
from jax import lax
from jax.experimental import pallas as pl
from jax.experimental.pallas import tpu as pltpu
from jax.experimental.pallas import tpu_sc as plsc

NUM_LAYERS = 3
EMB = 128
D_EDGE = 16
N_NODES = 10000
N_EDGES = 320000
N_GRAPHS = 256

NC = 2
NS = 16
NW = NC * NS
EDGES_PER_W = N_EDGES // NW
CHUNK = 80
NCHUNK = EDGES_PER_W // CHUNK
ROWS_MAIN = 624
TAIL = N_NODES - NS * ROWS_MAIN
ZROWS = 104


def _node_enc_body(x_ref, w_ref, b_ref, o_ref):
  o_ref[...] = (
      jnp.dot(x_ref[...], w_ref[...], preferred_element_type=jnp.float32)
      + b_ref[...]
  )


_node_enc = pl.pallas_call(
    _node_enc_body,
    out_shape=jax.ShapeDtypeStruct((N_NODES, EMB), jnp.float32),
)

EBLK = 8000


def _bond_enc_body(ea_ref, w_ref, b_ref, o_ref):
  o_ref[...] = (
      jnp.dot(ea_ref[...], w_ref[...], preferred_element_type=jnp.float32)
      + b_ref[...]
  )


_bond_enc = pl.pallas_call(
    _bond_enc_body,
    grid=(N_EDGES // EBLK,),
    in_specs=[
        pl.BlockSpec((EBLK, D_EDGE), lambda i: (i, 0)),
        pl.BlockSpec((D_EDGE, EMB), lambda i: (0, 0)),
        pl.BlockSpec((1, EMB), lambda i: (0, 0)),
    ],
    out_specs=pl.BlockSpec((EBLK, EMB), lambda i: (i, 0)),
    out_shape=jax.ShapeDtypeStruct((N_EDGES, EMB), jnp.float32),
)


def _bn_body(h_ref, g_ref, bt_ref, o_ref):
  h = h_ref[...]
  mu = jnp.mean(h, axis=0, keepdims=True)
  var = jnp.mean((h - mu) * (h - mu), axis=0, keepdims=True)
  o_ref[...] = (h - mu) * lax.rsqrt(var + 1e-5) * g_ref[...] + bt_ref[...]


_bn = pl.pallas_call(
    _bn_body,
    out_shape=jax.ShapeDtypeStruct((N_NODES, EMB), jnp.float32),
)

MBLK = 2000


def _mlp_body(hn_ref, a0_ref, a1_ref, w1_ref, b1_ref, w2_ref, b2_ref,
              eps_ref, o_ref, *, final):
  pre = (1.0 + eps_ref[0, 0]) * hn_ref[...] + a0_ref[...] + a1_ref[...]
  t = jnp.maximum(
      jnp.dot(pre, w1_ref[...], preferred_element_type=jnp.float32)
      + b1_ref[...], 0.0)
  out = (
      jnp.dot(t, w2_ref[...], preferred_element_type=jnp.float32)
      + b2_ref[...]
  )
  if not final:
    out = jnp.maximum(out, 0.0)
  o_ref[...] = out


def _make_mlp(final):
  return pl.pallas_call(
      functools.partial(_mlp_body, final=final),
      grid=(N_NODES // MBLK,),
      in_specs=[
          pl.BlockSpec((MBLK, EMB), lambda i: (i, 0)),
          pl.BlockSpec((MBLK, EMB), lambda i: (i, 0)),
          pl.BlockSpec((MBLK, EMB), lambda i: (i, 0)),
          pl.BlockSpec((EMB, 2 * EMB), lambda i: (0, 0)),
          pl.BlockSpec((1, 2 * EMB), lambda i: (0, 0)),
          pl.BlockSpec((2 * EMB, EMB), lambda i: (0, 0)),
          pl.BlockSpec((1, EMB), lambda i: (0, 0)),
          pl.BlockSpec((1, 1), lambda i: (0, 0)),
      ],
      out_specs=pl.BlockSpec((MBLK, EMB), lambda i: (i, 0)),
      out_shape=jax.ShapeDtypeStruct((N_NODES, EMB), jnp.float32),
  )


_mlp_mid = _make_mlp(final=False)
_mlp_last = _make_mlp(final=True)


def _pool_body(h_ref, b_ref, pw_ref, pb_ref, o_ref):
  score = jnp.dot(h_ref[...], pw_ref[...], preferred_element_type=jnp.float32)
  gids = lax.broadcasted_iota(jnp.int32, (1, N_GRAPHS), 1)
  onehot = (b_ref[...] == gids).astype(jnp.float32)
  preds = lax.dot_general(
      onehot, score, (((0,), (0,)), ((), ())),
      preferred_element_type=jnp.float32)
  o_ref[...] = preds + pb_ref[...]


_pool = pl.pallas_call(
    _pool_body,
    out_shape=jax.ShapeDtypeStruct((N_GRAPHS, 1), jnp.float32),
)


_sc_mesh = plsc.VectorSubcoreMesh(core_axis_name="c", subcore_axis_name="s")


@functools.partial(
    pl.kernel,
    out_type=jax.ShapeDtypeStruct((NC, N_NODES, EMB), jnp.float32),
    mesh=_sc_mesh,
    scratch_types=[
        pltpu.VMEM_SHARED((N_NODES, EMB), jnp.float32),
        pltpu.VMEM((CHUNK,), jnp.int32),
        pltpu.VMEM((CHUNK,), jnp.int32),
        pltpu.VMEM((CHUNK, EMB), jnp.float32),
        pltpu.VMEM((CHUNK, EMB), jnp.float32),
        pltpu.VMEM((ZROWS, EMB), jnp.float32),
        pltpu.SemaphoreType.DMA,
    ],
)
def _sc_message(hn_hbm, e_hbm, src_hbm, dst_hbm, out_hbm,
                agg_sh, src_v, dst_v, rows_v, e_v, zbuf, sem):
  c = lax.axis_index("c")
  s = lax.axis_index("s")

  zero16 = jnp.zeros((16,), jnp.float32)

  def _zrow(i, _):
    for k in range(EMB // 16):
      zbuf[i, pl.ds(k * 16, 16)] = zero16
    return 0

  lax.fori_loop(0, ZROWS, _zrow, 0)
  for j in range(ROWS_MAIN // ZROWS):
    pltpu.sync_copy(
        zbuf, agg_sh.at[pl.ds(s * ROWS_MAIN + j * ZROWS, ZROWS)])

  @pl.when(s == 0)
  def _zero_tail():
    pltpu.sync_copy(
        zbuf.at[pl.ds(0, TAIL)], agg_sh.at[pl.ds(NS * ROWS_MAIN, TAIL)])

  plsc.subcore_barrier()

  base = (c * NS + s) * EDGES_PER_W

  def _chunk(j, _):
    off = base + j * CHUNK
    pltpu.sync_copy(src_hbm.at[pl.ds(off, CHUNK)], src_v)
    pltpu.sync_copy(dst_hbm.at[pl.ds(off, CHUNK)], dst_v)
    pltpu.async_copy(hn_hbm.at[src_v], rows_v, sem).wait()
    pltpu.sync_copy(e_hbm.at[pl.ds(off, CHUNK)], e_v)

    def _edge(i, _):
      for k in range(EMB // 16):
        sl = pl.ds(k * 16, 16)
        rows_v[i, sl] = jnp.maximum(rows_v[i, sl] + e_v[i, sl], 0.0)
      return 0

    lax.fori_loop(0, CHUNK, _edge, 0)
    pltpu.sync_copy(rows_v, agg_sh.at[dst_v], add=True)
    return 0

  lax.fori_loop(0, NCHUNK, _chunk, 0)
  plsc.subcore_barrier()

  pltpu.sync_copy(
      agg_sh.at[pl.ds(s * ROWS_MAIN, ROWS_MAIN)],
      out_hbm.at[c, pl.ds(s * ROWS_MAIN, ROWS_MAIN)])

  @pl.when(s == 0)
  def _copy_tail():
    pltpu.sync_copy(
        agg_sh.at[pl.ds(NS * ROWS_MAIN, TAIL)],
        out_hbm.at[c, pl.ds(NS * ROWS_MAIN, TAIL)])


def kernel(x, edge_index, edge_attr, batch, node_W, node_b, bond_W, bond_b,
           conv_W1, conv_b1, conv_W2, conv_b2, eps, bn_gamma, bn_beta,
           pred_W, pred_b):
  h = _node_enc(x, node_W, node_b.reshape(1, EMB))
  e = _bond_enc(edge_attr, bond_W, bond_b.reshape(1, EMB))
  src = edge_index[0]
  dst = edge_index[1]
  for l in range(NUM_LAYERS):
    hn = _bn(h, bn_gamma[l].reshape(1, EMB), bn_beta[l].reshape(1, EMB))
    agg = _sc_message(hn, e, src, dst)
    mlp = _mlp_last if l == NUM_LAYERS - 1 else _mlp_mid
    h = mlp(hn, agg[0], agg[1], conv_W1[l], conv_b1[l].reshape(1, 2 * EMB),
            conv_W2[l], conv_b2[l].reshape(1, EMB), eps[l].reshape(1, 1))
  return _pool(h, batch.reshape(N_NODES, 1), pred_W, pred_b.reshape(1, 1))


# speedup vs baseline: 5.5600x; 2.0613x over previous
import functools

import jax
import jax.numpy as jnp
from jax import lax
from jax.experimental import pallas as pl
from jax.experimental.pallas import tpu as pltpu
from jax.experimental.pallas import tpu_sc as plsc

NUM_LAYERS = 3
EMB = 128
D_EDGE = 16
N_NODES = 10000
N_EDGES = 320000
N_GRAPHS = 256

NC = 2
NS = 16
NW = NC * NS
EDGES_PER_W = N_EDGES // NW
CHUNK = 40
NCHUNK = EDGES_PER_W // CHUNK
NBUF = 5
NEBUF = 2
NSLOT = 2 * NBUF
IDXPRE = 8
NGRP = NCHUNK // NSLOT
ROWS_MAIN = 624
TAIL = N_NODES - NS * ROWS_MAIN
ZROWS = 8


def _node_enc_body(x_ref, w_ref, b_ref, o_ref):
  o_ref[...] = (
      jnp.dot(x_ref[...], w_ref[...], preferred_element_type=jnp.float32)
      + b_ref[...]
  )


_node_enc = pl.pallas_call(
    _node_enc_body,
    out_shape=jax.ShapeDtypeStruct((N_NODES, EMB), jnp.float32),
)

EBLK = 8000


def _bond_enc_body(ea_ref, w_ref, b_ref, o_ref):
  o_ref[...] = (
      jnp.dot(ea_ref[...], w_ref[...], preferred_element_type=jnp.float32)
      + b_ref[...]
  )


_bond_enc = pl.pallas_call(
    _bond_enc_body,
    grid=(N_EDGES // EBLK,),
    in_specs=[
        pl.BlockSpec((EBLK, D_EDGE), lambda i: (i, 0)),
        pl.BlockSpec((D_EDGE, EMB), lambda i: (0, 0)),
        pl.BlockSpec((1, EMB), lambda i: (0, 0)),
    ],
    out_specs=pl.BlockSpec((EBLK, EMB), lambda i: (i, 0)),
    out_shape=jax.ShapeDtypeStruct((N_EDGES, EMB), jnp.float32),
)


def _bn_body(h_ref, g_ref, bt_ref, o_ref):
  h = h_ref[...]
  mu = jnp.mean(h, axis=0, keepdims=True)
  var = jnp.mean((h - mu) * (h - mu), axis=0, keepdims=True)
  o_ref[...] = (h - mu) * lax.rsqrt(var + 1e-5) * g_ref[...] + bt_ref[...]


_bn = pl.pallas_call(
    _bn_body,
    out_shape=jax.ShapeDtypeStruct((N_NODES, EMB), jnp.float32),
)

MBLK = 2000


def _mlp_body(hn_ref, a0_ref, a1_ref, w1_ref, b1_ref, w2_ref, b2_ref,
              eps_ref, o_ref, *, final):
  pre = (1.0 + eps_ref[0, 0]) * hn_ref[...] + a0_ref[...] + a1_ref[...]
  t = jnp.maximum(
      jnp.dot(pre, w1_ref[...], preferred_element_type=jnp.float32)
      + b1_ref[...], 0.0)
  out = (
      jnp.dot(t, w2_ref[...], preferred_element_type=jnp.float32)
      + b2_ref[...]
  )
  if not final:
    out = jnp.maximum(out, 0.0)
  o_ref[...] = out


def _make_mlp(final):
  return pl.pallas_call(
      functools.partial(_mlp_body, final=final),
      grid=(N_NODES // MBLK,),
      in_specs=[
          pl.BlockSpec((MBLK, EMB), lambda i: (i, 0)),
          pl.BlockSpec((MBLK, EMB), lambda i: (i, 0)),
          pl.BlockSpec((MBLK, EMB), lambda i: (i, 0)),
          pl.BlockSpec((EMB, 2 * EMB), lambda i: (0, 0)),
          pl.BlockSpec((1, 2 * EMB), lambda i: (0, 0)),
          pl.BlockSpec((2 * EMB, EMB), lambda i: (0, 0)),
          pl.BlockSpec((1, EMB), lambda i: (0, 0)),
          pl.BlockSpec((1, 1), lambda i: (0, 0)),
      ],
      out_specs=pl.BlockSpec((MBLK, EMB), lambda i: (i, 0)),
      out_shape=jax.ShapeDtypeStruct((N_NODES, EMB), jnp.float32),
  )


_mlp_mid = _make_mlp(final=False)
_mlp_last = _make_mlp(final=True)


def _pool_body(h_ref, b_ref, pw_ref, pb_ref, o_ref):
  score = jnp.dot(h_ref[...], pw_ref[...], preferred_element_type=jnp.float32)
  gids = lax.broadcasted_iota(jnp.int32, (1, N_GRAPHS), 1)
  onehot = (b_ref[...] == gids).astype(jnp.float32)
  preds = lax.dot_general(
      onehot, score, (((0,), (0,)), ((), ())),
      preferred_element_type=jnp.float32)
  o_ref[...] = preds + pb_ref[...]


_pool = pl.pallas_call(
    _pool_body,
    out_shape=jax.ShapeDtypeStruct((N_GRAPHS, 1), jnp.float32),
)


_sc_mesh = plsc.VectorSubcoreMesh(core_axis_name="c", subcore_axis_name="s")


@functools.partial(
    pl.kernel,
    out_type=jax.ShapeDtypeStruct((NC, N_NODES, EMB), jnp.float32),
    mesh=_sc_mesh,
    scratch_types=(
        [pltpu.VMEM_SHARED((N_NODES, EMB), jnp.float32)]
        + [pltpu.VMEM((CHUNK, EMB), jnp.float32)] * NBUF
        + [pltpu.VMEM((CHUNK, EMB), jnp.float32)] * NEBUF
        + [pltpu.VMEM((CHUNK,), jnp.int32)] * NSLOT
        + [pltpu.VMEM((CHUNK,), jnp.int32)] * NSLOT
        + [pltpu.VMEM((ZROWS, EMB), jnp.float32)]
        + [pltpu.SemaphoreType.DMA] * (2 * NBUF + NEBUF + NSLOT)
    ),
)
def _sc_message(hn_hbm, e_hbm, src_hbm, dst_hbm, out_hbm, agg_sh, *scr):
  rows = scr[0:NBUF]
  ebuf = scr[NBUF:NBUF + NEBUF]
  _o = NBUF + NEBUF
  isrc = scr[_o:_o + NSLOT]
  idst = scr[_o + NSLOT:_o + 2 * NSLOT]
  zbuf = scr[_o + 2 * NSLOT]
  _p = _o + 2 * NSLOT + 1
  insem = scr[_p:_p + NBUF]
  esem = scr[_p + NBUF:_p + NBUF + NEBUF]
  ssem = scr[_p + NBUF + NEBUF:_p + 2 * NBUF + NEBUF]
  isem = scr[_p + 2 * NBUF + NEBUF:_p + 2 * NBUF + NEBUF + NSLOT]

  c = lax.axis_index("c")
  s = lax.axis_index("s")
  w = c * NS + s
  base = w * EDGES_PER_W

  def _issue_idx(n, q):
    pltpu.async_copy(src_hbm.at[w, n], isrc[q], isem[q])
    pltpu.async_copy(dst_hbm.at[w, n], idst[q], isem[q])

  def _wait_idx(q):
    pltpu.make_async_copy(src_hbm.at[0, 0], isrc[q], isem[q]).wait()
    pltpu.make_async_copy(src_hbm.at[0, 0], idst[q], isem[q]).wait()

  def _issue_gather(b, q):
    pltpu.async_copy(hn_hbm.at[isrc[q]], rows[b], insem[b])

  def _issue_e(n, eb):
    pltpu.async_copy(
        e_hbm.at[pl.ds(base + n * CHUNK, CHUNK)], ebuf[eb], esem[eb])

  def _wait_in(b, eb):
    pltpu.make_async_copy(hn_hbm.at[isrc[0]], rows[b], insem[b]).wait()
    pltpu.make_async_copy(e_hbm.at[pl.ds(0, CHUNK)], ebuf[eb], esem[eb]).wait()

  def _wait_scat(b):
    pltpu.make_async_copy(rows[b], agg_sh.at[idst[0]], ssem[b]).wait()

  for n in range(IDXPRE):
    _issue_idx(n, n)
  for n in range(NBUF):
    _wait_idx(n)
    _issue_gather(n, n)
  for n in range(NEBUF):
    _issue_e(n, n)

  zero16 = jnp.zeros((16,), jnp.float32)

  @plsc.parallel_loop(0, ZROWS)
  def _zrow(i):
    for k in range(EMB // 16):
      zbuf[i, pl.ds(k * 16, 16)] = zero16

  def _zcopy(j, _):
    pltpu.sync_copy(
        zbuf, agg_sh.at[pl.ds(s * ROWS_MAIN + j * ZROWS, ZROWS)])
    return 0

  lax.fori_loop(0, ROWS_MAIN // ZROWS, _zcopy, 0)

  @pl.when(s == 0)
  def _zero_tail():
    for t in range(TAIL // ZROWS):
      pltpu.sync_copy(
          zbuf, agg_sh.at[pl.ds(NS * ROWS_MAIN + t * ZROWS, ZROWS)])

  plsc.subcore_barrier()

  def _grp(g, _):
    for u in range(NSLOT):
      jc = g * NSLOT + u
      b = u % NBUF
      eb = u % NEBUF
      _wait_in(b, eb)

      @plsc.parallel_loop(0, CHUNK, unroll=2)
      def _edge(i):
        for k in range(EMB // 16):
          sl = pl.ds(k * 16, 16)
          rows[b][i, sl] = jnp.maximum(rows[b][i, sl] + ebuf[eb][i, sl], 0.0)

      pltpu.async_copy(rows[b], agg_sh.at[idst[u]], ssem[b], add=True)

      @pl.when(jc + NEBUF < NCHUNK)
      def _pre_e():
        _issue_e(jc + NEBUF, eb)

      @pl.when(jc + IDXPRE < NCHUNK)
      def _pre_idx():
        _issue_idx(jc + IDXPRE, (u + IDXPRE) % NSLOT)

      bp = (b + NBUF - 1) % NBUF
      qp = (u + NBUF - 1) % NSLOT
      jp = jc + NBUF - 1

      @pl.when(jnp.logical_and(jp >= NBUF, jp < NCHUNK))
      def _refill():
        _wait_scat(bp)
        _wait_idx(qp)
        _issue_gather(bp, qp)

    return 0

  lax.fori_loop(0, NGRP, _grp, 0)

  for b in range(NBUF):
    _wait_scat(b)
  plsc.subcore_barrier()

  pltpu.sync_copy(
      agg_sh.at[pl.ds(s * ROWS_MAIN, ROWS_MAIN)],
      out_hbm.at[c, pl.ds(s * ROWS_MAIN, ROWS_MAIN)])

  @pl.when(s == 0)
  def _copy_tail():
    pltpu.sync_copy(
        agg_sh.at[pl.ds(NS * ROWS_MAIN, TAIL)],
        out_hbm.at[c, pl.ds(NS * ROWS_MAIN, TAIL)])


def kernel(x, edge_index, edge_attr, batch, node_W, node_b, bond_W, bond_b,
           conv_W1, conv_b1, conv_W2, conv_b2, eps, bn_gamma, bn_beta,
           pred_W, pred_b):
  h = _node_enc(x, node_W, node_b.reshape(1, EMB))
  e = _bond_enc(edge_attr, bond_W, bond_b.reshape(1, EMB))
  src = edge_index[0].reshape(NW, NCHUNK, CHUNK)
  dst = edge_index[1].reshape(NW, NCHUNK, CHUNK)
  for l in range(NUM_LAYERS):
    hn = _bn(h, bn_gamma[l].reshape(1, EMB), bn_beta[l].reshape(1, EMB))
    agg = _sc_message(hn, e, src, dst)
    mlp = _mlp_last if l == NUM_LAYERS - 1 else _mlp_mid
    h = mlp(hn, agg[0], agg[1], conv_W1[l], conv_b1[l].reshape(1, 2 * EMB),
            conv_W2[l], conv_b2[l].reshape(1, EMB), eps[l].reshape(1, 1))
  return _pool(h, batch.reshape(N_NODES, 1), pred_W, pred_b.reshape(1, 1))


# speedup vs baseline: 5.6067x; 1.0084x over previous
import functools

import jax
import jax.numpy as jnp
from jax import lax
from jax.experimental import pallas as pl
from jax.experimental.pallas import tpu as pltpu
from jax.experimental.pallas import tpu_sc as plsc

NUM_LAYERS = 3
EMB = 128
D_EDGE = 16
N_NODES = 10000
N_EDGES = 320000
N_GRAPHS = 256

NC = 2
NS = 16
NW = NC * NS
EDGES_PER_W = N_EDGES // NW
CHUNK = 40
NCHUNK = EDGES_PER_W // CHUNK
NBUF = 5
NEBUF = 2
NSLOT = 2 * NBUF
IDXPRE = 8
NGRP = NCHUNK // NSLOT
ROWS_MAIN = 624
TAIL = N_NODES - NS * ROWS_MAIN
ZROWS = 8


def _node_enc_body(x_ref, w_ref, b_ref, o_ref):
  o_ref[...] = (
      jnp.dot(x_ref[...], w_ref[...], preferred_element_type=jnp.float32)
      + b_ref[...]
  )


_node_enc = pl.pallas_call(
    _node_enc_body,
    out_shape=jax.ShapeDtypeStruct((N_NODES, EMB), jnp.float32),
)

EBLK = 8000


def _bond_enc_body(ea_ref, w_ref, b_ref, o_ref):
  o_ref[...] = (
      jnp.dot(ea_ref[...], w_ref[...], preferred_element_type=jnp.float32)
      + b_ref[...]
  )


_bond_enc = pl.pallas_call(
    _bond_enc_body,
    grid=(N_EDGES // EBLK,),
    in_specs=[
        pl.BlockSpec((EBLK, D_EDGE), lambda i: (i, 0)),
        pl.BlockSpec((D_EDGE, EMB), lambda i: (0, 0)),
        pl.BlockSpec((1, EMB), lambda i: (0, 0)),
    ],
    out_specs=pl.BlockSpec((EBLK, EMB), lambda i: (i, 0)),
    out_shape=jax.ShapeDtypeStruct((N_EDGES, EMB), jnp.float32),
)


def _bn_body(h_ref, g_ref, bt_ref, o_ref):
  h = h_ref[...]
  mu = jnp.mean(h, axis=0, keepdims=True)
  var = jnp.mean((h - mu) * (h - mu), axis=0, keepdims=True)
  o_ref[...] = (h - mu) * lax.rsqrt(var + 1e-5) * g_ref[...] + bt_ref[...]


_bn = pl.pallas_call(
    _bn_body,
    out_shape=jax.ShapeDtypeStruct((N_NODES, EMB), jnp.float32),
)

MBLK = 2000


def _mlp_body(hn_ref, a0_ref, a1_ref, w1_ref, b1_ref, w2_ref, b2_ref,
              eps_ref, o_ref, *, final):
  pre = (1.0 + eps_ref[0, 0]) * hn_ref[...] + a0_ref[...] + a1_ref[...]
  t = jnp.maximum(
      jnp.dot(pre, w1_ref[...], preferred_element_type=jnp.float32)
      + b1_ref[...], 0.0)
  out = (
      jnp.dot(t, w2_ref[...], preferred_element_type=jnp.float32)
      + b2_ref[...]
  )
  if not final:
    out = jnp.maximum(out, 0.0)
  o_ref[...] = out


def _make_mlp(final):
  return pl.pallas_call(
      functools.partial(_mlp_body, final=final),
      grid=(N_NODES // MBLK,),
      in_specs=[
          pl.BlockSpec((MBLK, EMB), lambda i: (i, 0)),
          pl.BlockSpec((MBLK, EMB), lambda i: (i, 0)),
          pl.BlockSpec((MBLK, EMB), lambda i: (i, 0)),
          pl.BlockSpec((EMB, 2 * EMB), lambda i: (0, 0)),
          pl.BlockSpec((1, 2 * EMB), lambda i: (0, 0)),
          pl.BlockSpec((2 * EMB, EMB), lambda i: (0, 0)),
          pl.BlockSpec((1, EMB), lambda i: (0, 0)),
          pl.BlockSpec((1, 1), lambda i: (0, 0)),
      ],
      out_specs=pl.BlockSpec((MBLK, EMB), lambda i: (i, 0)),
      out_shape=jax.ShapeDtypeStruct((N_NODES, EMB), jnp.float32),
  )


_mlp_mid = _make_mlp(final=False)
_mlp_last = _make_mlp(final=True)


def _pool_body(h_ref, b_ref, pw_ref, pb_ref, o_ref):
  score = jnp.dot(h_ref[...], pw_ref[...], preferred_element_type=jnp.float32)
  gids = lax.broadcasted_iota(jnp.int32, (1, N_GRAPHS), 1)
  onehot = (b_ref[...] == gids).astype(jnp.float32)
  preds = lax.dot_general(
      onehot, score, (((0,), (0,)), ((), ())),
      preferred_element_type=jnp.float32)
  o_ref[...] = preds + pb_ref[...]


_pool = pl.pallas_call(
    _pool_body,
    out_shape=jax.ShapeDtypeStruct((N_GRAPHS, 1), jnp.float32),
)


_sc_mesh = plsc.VectorSubcoreMesh(core_axis_name="c", subcore_axis_name="s")


@functools.partial(
    pl.kernel,
    out_type=jax.ShapeDtypeStruct((NC, N_NODES, EMB), jnp.float32),
    mesh=_sc_mesh,
    scratch_types=(
        [pltpu.VMEM_SHARED((N_NODES, EMB), jnp.float32)]
        + [pltpu.VMEM((CHUNK, EMB), jnp.float32)] * NBUF
        + [pltpu.VMEM((CHUNK, EMB), jnp.float32)] * NEBUF
        + [pltpu.VMEM((CHUNK,), jnp.int32)] * NSLOT
        + [pltpu.VMEM((CHUNK,), jnp.int32)] * NSLOT
        + [pltpu.VMEM((ZROWS, EMB), jnp.float32)]
        + [pltpu.SemaphoreType.DMA] * (2 * NBUF + NEBUF + NSLOT)
    ),
)
def _sc_message(hn_hbm, e_hbm, src_hbm, dst_hbm, out_hbm, agg_sh, *scr):
  rows = scr[0:NBUF]
  ebuf = scr[NBUF:NBUF + NEBUF]
  _o = NBUF + NEBUF
  isrc = scr[_o:_o + NSLOT]
  idst = scr[_o + NSLOT:_o + 2 * NSLOT]
  zbuf = scr[_o + 2 * NSLOT]
  _p = _o + 2 * NSLOT + 1
  insem = scr[_p:_p + NBUF]
  esem = scr[_p + NBUF:_p + NBUF + NEBUF]
  ssem = scr[_p + NBUF + NEBUF:_p + 2 * NBUF + NEBUF]
  isem = scr[_p + 2 * NBUF + NEBUF:_p + 2 * NBUF + NEBUF + NSLOT]

  c = lax.axis_index("c")
  s = lax.axis_index("s")
  w = c * NS + s
  base = w * EDGES_PER_W

  def _issue_idx(n, q):
    pltpu.async_copy(src_hbm.at[w, n], isrc[q], isem[q])
    pltpu.async_copy(dst_hbm.at[w, n], idst[q], isem[q])

  def _wait_idx(q):
    pltpu.make_async_copy(src_hbm.at[0, 0], isrc[q], isem[q]).wait()
    pltpu.make_async_copy(src_hbm.at[0, 0], idst[q], isem[q]).wait()

  def _issue_gather(b, q):
    pltpu.async_copy(hn_hbm.at[isrc[q]], rows[b], insem[b])

  def _issue_e(n, eb):
    pltpu.async_copy(
        e_hbm.at[pl.ds(base + n * CHUNK, CHUNK)], ebuf[eb], esem[eb])

  def _wait_in(b, eb):
    pltpu.make_async_copy(hn_hbm.at[isrc[0]], rows[b], insem[b]).wait()
    pltpu.make_async_copy(e_hbm.at[pl.ds(0, CHUNK)], ebuf[eb], esem[eb]).wait()

  def _wait_scat(b):
    pltpu.make_async_copy(rows[b], agg_sh.at[idst[0]], ssem[b]).wait()

  for n in range(IDXPRE):
    _issue_idx(n, n)
  for n in range(NBUF):
    _wait_idx(n)
    _issue_gather(n, n)
  for n in range(NEBUF):
    _issue_e(n, n)

  zero16 = jnp.zeros((16,), jnp.float32)

  @plsc.parallel_loop(0, ZROWS)
  def _zrow(i):
    for k in range(EMB // 16):
      zbuf[i, pl.ds(k * 16, 16)] = zero16

  def _zcopy(j, _):
    pltpu.sync_copy(
        zbuf, agg_sh.at[pl.ds(s * ROWS_MAIN + j * ZROWS, ZROWS)])
    return 0

  lax.fori_loop(0, ROWS_MAIN // ZROWS, _zcopy, 0)

  @pl.when(s == 0)
  def _zero_tail():
    for t in range(TAIL // ZROWS):
      pltpu.sync_copy(
          zbuf, agg_sh.at[pl.ds(NS * ROWS_MAIN + t * ZROWS, ZROWS)])

  plsc.subcore_barrier()

  def _grp(g, _):
    for u in range(NSLOT):
      jc = g * NSLOT + u
      b = u % NBUF
      eb = u % NEBUF
      _wait_in(b, eb)

      @plsc.parallel_loop(0, CHUNK, unroll=4)
      def _edge(i):
        for k in range(EMB // 16):
          sl = pl.ds(k * 16, 16)
          rows[b][i, sl] = jnp.maximum(rows[b][i, sl] + ebuf[eb][i, sl], 0.0)

      pltpu.async_copy(rows[b], agg_sh.at[idst[u]], ssem[b], add=True)

      @pl.when(jc + NEBUF < NCHUNK)
      def _pre_e():
        _issue_e(jc + NEBUF, eb)

      @pl.when(jc + IDXPRE < NCHUNK)
      def _pre_idx():
        _issue_idx(jc + IDXPRE, (u + IDXPRE) % NSLOT)

      bp = (b + NBUF - 1) % NBUF
      qp = (u + NBUF - 1) % NSLOT
      jp = jc + NBUF - 1

      @pl.when(jnp.logical_and(jp >= NBUF, jp < NCHUNK))
      def _refill():
        _wait_scat(bp)
        _wait_idx(qp)
        _issue_gather(bp, qp)

    return 0

  lax.fori_loop(0, NGRP, _grp, 0)

  for b in range(NBUF):
    _wait_scat(b)
  plsc.subcore_barrier()

  pltpu.sync_copy(
      agg_sh.at[pl.ds(s * ROWS_MAIN, ROWS_MAIN)],
      out_hbm.at[c, pl.ds(s * ROWS_MAIN, ROWS_MAIN)])

  @pl.when(s == 0)
  def _copy_tail():
    pltpu.sync_copy(
        agg_sh.at[pl.ds(NS * ROWS_MAIN, TAIL)],
        out_hbm.at[c, pl.ds(NS * ROWS_MAIN, TAIL)])


def kernel(x, edge_index, edge_attr, batch, node_W, node_b, bond_W, bond_b,
           conv_W1, conv_b1, conv_W2, conv_b2, eps, bn_gamma, bn_beta,
           pred_W, pred_b):
  h = _node_enc(x, node_W, node_b.reshape(1, EMB))
  e = _bond_enc(edge_attr, bond_W, bond_b.reshape(1, EMB))
  src = edge_index[0].reshape(NW, NCHUNK, CHUNK)
  dst = edge_index[1].reshape(NW, NCHUNK, CHUNK)
  for l in range(NUM_LAYERS):
    hn = _bn(h, bn_gamma[l].reshape(1, EMB), bn_beta[l].reshape(1, EMB))
    agg = _sc_message(hn, e, src, dst)
    mlp = _mlp_last if l == NUM_LAYERS - 1 else _mlp_mid
    h = mlp(hn, agg[0], agg[1], conv_W1[l], conv_b1[l].reshape(1, 2 * EMB),
            conv_W2[l], conv_b2[l].reshape(1, EMB), eps[l].reshape(1, 1))
  return _pool(h, batch.reshape(N_NODES, 1), pred_W, pred_b.reshape(1, 1))
